# baseline (speedup 1.0000x reference)
import functools

import jax
import jax.numpy as jnp
from jax import lax
from jax.experimental import pallas as pl
from jax.experimental.pallas import tpu as pltpu
from jax.experimental.pallas import tpu_sc as plsc

_B = 16384
_D = 32
_NW = 32
_BPW = _B // _NW
_CHUNK = 128
_NCH = _BPW // _CHUNK


def _body(users_hbm, items_hbm, ut_hbm, it_hbm, out_hbm,
          uidx, iidx, urows, irows, outv, sem):
    wid = lax.axis_index("s") * 2 + lax.axis_index("c")
    base = wid * _BPW

    pltpu.sync_copy(users_hbm.at[pl.ds(base, _BPW)], uidx)
    pltpu.sync_copy(items_hbm.at[pl.ds(base, _BPW)], iidx)

    copies = []
    for c in range(_NCH):
        sl = pl.ds(c * _CHUNK, _CHUNK)
        copies.append(pltpu.async_copy(ut_hbm.at[uidx.at[sl]], urows.at[sl], sem))
        copies.append(pltpu.async_copy(it_hbm.at[iidx.at[sl]], irows.at[sl], sem))
    for h in copies:
        h.wait()

    lane = lax.iota(jnp.int32, 16)
    perms = {d: lane ^ d for d in (8, 4, 2, 1)}

    def block(b, _):
        s = []
        for rr in range(16):
            r = b * 16 + rr
            s.append(urows[r, pl.ds(0, 16)] * irows[r, pl.ds(0, 16)]
                     + urows[r, pl.ds(16, 16)] * irows[r, pl.ds(16, 16)])
        for d in (8, 4, 2, 1):
            h = [v + v.at[perms[d]].get(mode="promise_in_bounds")
                 for v in s]
            half = len(h) // 2
            s = [jnp.where((lane & d) == 0, h[j], h[j + half])
                 for j in range(half)]
        outv[pl.ds(b * 16, 16)] = 1.0 / (1.0 + jnp.exp(-s[0]))
        return 0

    lax.fori_loop(0, _BPW // 16, block, 0)

    pltpu.sync_copy(outv, out_hbm.at[pl.ds(base, _BPW)])


@jax.jit
def _run(users, items, user_table, item_table):
    mesh = plsc.VectorSubcoreMesh(core_axis_name="c", subcore_axis_name="s")
    k = pl.kernel(
        _body,
        out_type=jax.ShapeDtypeStruct((_B,), jnp.float32),
        mesh=mesh,
        scratch_types=[
            pltpu.VMEM((_BPW,), jnp.int32),
            pltpu.VMEM((_BPW,), jnp.int32),
            pltpu.VMEM((_BPW, _D), jnp.float32),
            pltpu.VMEM((_BPW, _D), jnp.float32),
            pltpu.VMEM((_BPW,), jnp.float32),
            pltpu.SemaphoreType.DMA,
        ],
        compiler_params=pltpu.CompilerParams(use_tc_tiling_on_sc=False),
    )
    return k(users, items, user_table, item_table)


def kernel(users, items, user_table, item_table):
    return _run(users, items, user_table, item_table)
